# speedup vs baseline: 5.8938x; 5.8938x over previous
import functools

import jax
import jax.numpy as jnp
from jax import lax
from jax.experimental import pallas as pl
from jax.experimental.pallas import tpu as pltpu
from jax.experimental.pallas import tpu_sc as plsc

_NC = 2
_NS = 16
_NW = _NC * _NS

_IDXW = 128
_K = 2
_CHUNK = _K * _IDXW


def _scale_table_body(w_ref, o_ref):
    o_ref[...] = jnp.maximum(w_ref[...], 0.0) * o_ref.shape[-1] ** 0.5


def _scaled_table(w):
    v, d = w.shape
    bs = 1024
    grid = (v + bs - 1) // bs
    return pl.pallas_call(
        _scale_table_body,
        grid=(grid,),
        in_specs=[pl.BlockSpec((bs, d), lambda i: (i, 0))],
        out_specs=pl.BlockSpec((bs, d), lambda i: (i, 0)),
        out_shape=jax.ShapeDtypeStruct((v, d), w.dtype),
    )(w)


@functools.partial(jax.jit, static_argnums=(2, 3))
def _sc_gather(table, idx2, n_rows, d):
    rows_per_w = n_rows // _NW
    n_chunks = rows_per_w // _CHUNK
    irows_per_w = rows_per_w // _IDXW

    mesh = plsc.VectorSubcoreMesh(core_axis_name="c", subcore_axis_name="s")

    @functools.partial(
        pl.kernel,
        mesh=mesh,
        out_type=jax.ShapeDtypeStruct((n_rows, d), jnp.float32),
        scratch_types=[
            pltpu.VMEM((_K, _IDXW), jnp.int32),
            pltpu.VMEM((_CHUNK, d), jnp.float32),
            pltpu.SemaphoreType.DMA,
        ],
    )
    def k(table_hbm, idx_hbm, out_hbm, idx_v, rows_v, sem):
        wid = lax.axis_index("s") * _NC + lax.axis_index("c")
        irow0 = wid * irows_per_w

        def chunk(c, carry):
            base = irow0 + c * _K
            pltpu.sync_copy(idx_hbm.at[pl.ds(base, _K)], idx_v)
            cps = [
                pltpu.async_copy(
                    table_hbm.at[idx_v.at[j]],
                    rows_v.at[pl.ds(j * _IDXW, _IDXW)],
                    sem,
                )
                for j in range(_K)
            ]
            for cp in cps:
                cp.wait()
            pltpu.sync_copy(rows_v, out_hbm.at[pl.ds(base * _IDXW, _CHUNK)])
            return carry

        lax.fori_loop(0, n_chunks, chunk, 0)

    return k(table, idx2)


def kernel(x, embed_weight):
    b, h = x.shape
    v, d = embed_weight.shape
    n_rows = b * h
    table2 = _scaled_table(embed_weight)
    idx2 = x.reshape(n_rows // _IDXW, _IDXW).astype(jnp.int32)
    out = _sc_gather(table2, idx2, n_rows, d)
    return out.reshape(b, h, d)


# speedup vs baseline: 7.5346x; 1.2784x over previous
import functools

import jax
import jax.numpy as jnp
from jax import lax
from jax.experimental import pallas as pl
from jax.experimental.pallas import tpu as pltpu
from jax.experimental.pallas import tpu_sc as plsc

_NC = 2
_NS = 16
_NW = _NC * _NS

_IDXW = 128
_K = 2
_CHUNK = _K * _IDXW


def _scale_table_body(w_ref, o_ref):
    o_ref[...] = jnp.maximum(w_ref[...], 0.0) * o_ref.shape[-1] ** 0.5


def _scaled_table(w):
    v, d = w.shape
    bs = 1024
    grid = (v + bs - 1) // bs
    return pl.pallas_call(
        _scale_table_body,
        grid=(grid,),
        in_specs=[pl.BlockSpec((bs, d), lambda i: (i, 0))],
        out_specs=pl.BlockSpec((bs, d), lambda i: (i, 0)),
        out_shape=jax.ShapeDtypeStruct((v, d), w.dtype),
    )(w)


@functools.partial(jax.jit, static_argnums=(2, 3))
def _sc_gather(table, idx2, n_rows, d):
    rows_per_w = n_rows // _NW
    n_chunks = rows_per_w // _CHUNK
    irows_per_w = rows_per_w // _IDXW
    assert n_chunks % 2 == 0 and n_chunks >= 4

    mesh = plsc.VectorSubcoreMesh(core_axis_name="c", subcore_axis_name="s")

    @functools.partial(
        pl.kernel,
        mesh=mesh,
        out_type=jax.ShapeDtypeStruct((n_rows, d), jnp.float32),
        scratch_types=[
            pltpu.VMEM((irows_per_w, _IDXW), jnp.int32),
            pltpu.VMEM((2, _CHUNK, d), jnp.float32),
            pltpu.SemaphoreType.DMA,
            pltpu.SemaphoreType.DMA,
            pltpu.SemaphoreType.DMA,
            pltpu.SemaphoreType.DMA,
        ],
    )
    def k(table_hbm, idx_hbm, out_hbm, idx_all, rows_v, g_a, g_b, o_a, o_b):
        wid = lax.axis_index("s") * _NC + lax.axis_index("c")
        irow0 = wid * irows_per_w

        pltpu.sync_copy(idx_hbm.at[pl.ds(irow0, irows_per_w)], idx_all)

        def fire_gather(b, sem, c):
            for j in range(_K):
                pltpu.async_copy(
                    table_hbm.at[idx_all.at[c * _K + j]],
                    rows_v.at[b, pl.ds(j * _IDXW, _IDXW)],
                    sem,
                )

        def wait_gather(b, sem):
            for j in range(_K):
                pltpu.make_async_copy(
                    table_hbm.at[idx_all.at[0]],
                    rows_v.at[b, pl.ds(j * _IDXW, _IDXW)],
                    sem,
                ).wait()

        def fire_out(b, sem, c):
            pltpu.async_copy(
                rows_v.at[b],
                out_hbm.at[pl.ds((irow0 + c * _K) * _IDXW, _CHUNK)],
                sem,
            )

        def wait_out(b, sem):
            pltpu.make_async_copy(
                rows_v.at[b],
                out_hbm.at[pl.ds(0, _CHUNK)],
                sem,
            ).wait()

        fire_gather(0, g_a, 0)
        fire_gather(1, g_b, 1)
        wait_gather(0, g_a)
        fire_out(0, o_a, 0)

        def body(i, carry):
            wait_out(0, o_a)
            fire_gather(0, g_a, 2 * i + 2)
            wait_gather(1, g_b)
            fire_out(1, o_b, 2 * i + 1)
            wait_out(1, o_b)
            fire_gather(1, g_b, 2 * i + 3)
            wait_gather(0, g_a)
            fire_out(0, o_a, 2 * i + 2)
            return carry

        lax.fori_loop(0, n_chunks // 2 - 1, body, 0)

        wait_gather(1, g_b)
        fire_out(1, o_b, n_chunks - 1)
        wait_out(1, o_b)
        wait_out(0, o_a)

    return k(table, idx2)


def kernel(x, embed_weight):
    b, h = x.shape
    v, d = embed_weight.shape
    n_rows = b * h
    table2 = _scaled_table(embed_weight)
    idx2 = x.reshape(n_rows // _IDXW, _IDXW).astype(jnp.int32)
    out = _sc_gather(table2, idx2, n_rows, d)
    return out.reshape(b, h, d)


# speedup vs baseline: 7.5386x; 1.0005x over previous
import functools

import jax
import jax.numpy as jnp
from jax import lax
from jax.experimental import pallas as pl
from jax.experimental.pallas import tpu as pltpu
from jax.experimental.pallas import tpu_sc as plsc

_NC = 2
_NS = 16
_NW = _NC * _NS

_IDXW = 128
_NBUF = 4
_DELAY = 2


def _scale_table_body(w_ref, o_ref):
    o_ref[...] = jnp.maximum(w_ref[...], 0.0) * o_ref.shape[-1] ** 0.5


def _scaled_table(w):
    v, d = w.shape
    bs = 1024
    grid = (v + bs - 1) // bs
    return pl.pallas_call(
        _scale_table_body,
        grid=(grid,),
        in_specs=[pl.BlockSpec((bs, d), lambda i: (i, 0))],
        out_specs=pl.BlockSpec((bs, d), lambda i: (i, 0)),
        out_shape=jax.ShapeDtypeStruct((v, d), w.dtype),
    )(w)


@functools.partial(jax.jit, static_argnums=(2, 3))
def _sc_gather(table, idx2, n_rows, d):
    rows_per_w = n_rows // _NW
    n_chunks = rows_per_w // _IDXW
    irows_per_w = n_chunks
    assert n_chunks % _NBUF == 0 and n_chunks >= 2 * _NBUF

    mesh = plsc.VectorSubcoreMesh(core_axis_name="c", subcore_axis_name="s")

    sems = [pltpu.SemaphoreType.DMA] * (2 * _NBUF)

    @functools.partial(
        pl.kernel,
        mesh=mesh,
        out_type=jax.ShapeDtypeStruct((n_rows, d), jnp.float32),
        scratch_types=[
            pltpu.VMEM((irows_per_w, _IDXW), jnp.int32),
            pltpu.VMEM((_NBUF, _IDXW, d), jnp.float32),
        ] + sems,
    )
    def k(table_hbm, idx_hbm, out_hbm, idx_all, rows_v, *all_sems):
        g_sem = all_sems[:_NBUF]
        o_sem = all_sems[_NBUF:]
        wid = lax.axis_index("s") * _NC + lax.axis_index("c")
        irow0 = wid * irows_per_w

        pltpu.sync_copy(idx_hbm.at[pl.ds(irow0, irows_per_w)], idx_all)

        def fire_gather(b, c):
            pltpu.async_copy(
                table_hbm.at[idx_all.at[c]], rows_v.at[b], g_sem[b]
            )

        def wait_gather(b):
            pltpu.make_async_copy(
                table_hbm.at[idx_all.at[0]], rows_v.at[b], g_sem[b]
            ).wait()

        def fire_out(b, c):
            pltpu.async_copy(
                rows_v.at[b],
                out_hbm.at[pl.ds((irow0 + c) * _IDXW, _IDXW)],
                o_sem[b],
            )

        def wait_out(b):
            pltpu.make_async_copy(
                rows_v.at[b], out_hbm.at[pl.ds(0, _IDXW)], o_sem[b]
            ).wait()

        def step(c, k_static, fire_g, wait_g, wait_o):
            if wait_o:
                wait_out(k_static)
            if fire_g:
                fire_gather(k_static, c)
            if wait_g:
                b2 = (k_static - _DELAY) % _NBUF
                wait_gather(b2)
                fire_out(b2, c - _DELAY)

        for c in range(_NBUF):
            step(c, c, True, c >= _DELAY, False)

        def body(i, carry):
            for kk in range(_NBUF):
                step(_NBUF + i * _NBUF + kk, kk, True, True, True)
            return carry

        lax.fori_loop(0, n_chunks // _NBUF - 1, body, 0)

        for c in range(n_chunks, n_chunks + _DELAY):
            step(c, c % _NBUF, False, True, False)
        for b in range(_NBUF):
            wait_out(b)

    return k(table, idx2)


def kernel(x, embed_weight):
    b, h = x.shape
    v, d = embed_weight.shape
    n_rows = b * h
    table2 = _scaled_table(embed_weight)
    idx2 = x.reshape(n_rows // _IDXW, _IDXW).astype(jnp.int32)
    out = _sc_gather(table2, idx2, n_rows, d)
    return out.reshape(b, h, d)


# speedup vs baseline: 9.1978x; 1.2201x over previous
import functools

import jax
import jax.numpy as jnp
from jax import lax
from jax.experimental import pallas as pl
from jax.experimental.pallas import tpu as pltpu
from jax.experimental.pallas import tpu_sc as plsc

_NC = 2
_NS = 16
_NW = _NC * _NS

_IDXW = 128
_NBUF = 4
_DELAY = 2


def _scale_table_body(w_ref, o_ref):
    o_ref[...] = jnp.maximum(w_ref[...], 0.0) * o_ref.shape[-1] ** 0.5


def _scaled_table(w):
    v, d = w.shape
    bs = 1024
    grid = (v + bs - 1) // bs
    return pl.pallas_call(
        _scale_table_body,
        grid=(grid,),
        in_specs=[pl.BlockSpec((bs, d), lambda i: (i, 0))],
        out_specs=pl.BlockSpec((bs, d), lambda i: (i, 0)),
        out_shape=jax.ShapeDtypeStruct((v, d), w.dtype),
    )(w)


@functools.partial(jax.jit, static_argnums=(2, 3))
def _sc_gather(table, idx2, n_rows, d):
    rows_per_w = n_rows // _NW
    n_chunks = rows_per_w // _IDXW
    irows_per_w = n_chunks
    assert n_chunks % _NBUF == 0 and n_chunks >= 2 * _NBUF

    mesh = plsc.VectorSubcoreMesh(core_axis_name="c", subcore_axis_name="s")

    sems = [pltpu.SemaphoreType.DMA] * (2 * _NBUF)

    @functools.partial(
        pl.kernel,
        mesh=mesh,
        out_type=jax.ShapeDtypeStruct((n_rows, d), jnp.float32),
        scratch_types=[
            pltpu.VMEM((irows_per_w, _IDXW), jnp.int32),
            pltpu.VMEM((_NBUF, _IDXW, d), jnp.float32),
        ] + sems,
    )
    def k(table_hbm, idx_hbm, out_hbm, idx_all, rows_v, *all_sems):
        g_sem = all_sems[:_NBUF]
        o_sem = all_sems[_NBUF:]
        wid = lax.axis_index("s") * _NC + lax.axis_index("c")
        irow0 = wid * irows_per_w

        pltpu.sync_copy(idx_hbm.at[pl.ds(irow0, irows_per_w)], idx_all)

        def fire_gather(b, c):
            pltpu.async_copy(
                table_hbm.at[idx_all.at[c]], rows_v.at[b], g_sem[b]
            )

        def wait_gather(b):
            pltpu.make_async_copy(
                table_hbm.at[idx_all.at[0]], rows_v.at[b], g_sem[b]
            ).wait()

        def fire_out(b, c):
            pltpu.async_copy(
                rows_v.at[b],
                out_hbm.at[pl.ds((irow0 + c) * _IDXW, _IDXW)],
                o_sem[b],
            )

        def wait_out(b):
            pltpu.make_async_copy(
                rows_v.at[b], out_hbm.at[pl.ds(0, _IDXW)], o_sem[b]
            ).wait()

        scale = float(d) ** 0.5

        def relu_scale(b):
            def row_fn(r, carry):
                for j in range(d // 16):
                    v = rows_v[b, r, pl.ds(j * 16, 16)]
                    rows_v[b, r, pl.ds(j * 16, 16)] = (
                        jnp.maximum(v, 0.0) * scale
                    )
                return carry

            lax.fori_loop(0, _IDXW, row_fn, 0)

        def step(c, k_static, fire_g, wait_g, wait_o):
            if wait_o:
                wait_out(k_static)
            if fire_g:
                fire_gather(k_static, c)
            if wait_g:
                b2 = (k_static - _DELAY) % _NBUF
                wait_gather(b2)
                relu_scale(b2)
                fire_out(b2, c - _DELAY)

        for c in range(_NBUF):
            step(c, c, True, c >= _DELAY, False)

        def body(i, carry):
            for kk in range(_NBUF):
                step(_NBUF + i * _NBUF + kk, kk, True, True, True)
            return carry

        lax.fori_loop(0, n_chunks // _NBUF - 1, body, 0)

        for c in range(n_chunks, n_chunks + _DELAY):
            step(c, c % _NBUF, False, True, False)
        for b in range(_NBUF):
            wait_out(b)

    return k(table, idx2)


def kernel(x, embed_weight):
    b, h = x.shape
    v, d = embed_weight.shape
    n_rows = b * h
    idx2 = x.reshape(n_rows // _IDXW, _IDXW).astype(jnp.int32)
    out = _sc_gather(embed_weight, idx2, n_rows, d)
    return out.reshape(b, h, d)


# speedup vs baseline: 9.2244x; 1.0029x over previous
import functools

import jax
import jax.numpy as jnp
from jax import lax
from jax.experimental import pallas as pl
from jax.experimental.pallas import tpu as pltpu
from jax.experimental.pallas import tpu_sc as plsc

_NC = 2
_NS = 16
_NW = _NC * _NS

_IDXW = 128
_NBUF = 5
_DELAY = 2


def _scale_table_body(w_ref, o_ref):
    o_ref[...] = jnp.maximum(w_ref[...], 0.0) * o_ref.shape[-1] ** 0.5


def _scaled_table(w):
    v, d = w.shape
    bs = 1024
    grid = (v + bs - 1) // bs
    return pl.pallas_call(
        _scale_table_body,
        grid=(grid,),
        in_specs=[pl.BlockSpec((bs, d), lambda i: (i, 0))],
        out_specs=pl.BlockSpec((bs, d), lambda i: (i, 0)),
        out_shape=jax.ShapeDtypeStruct((v, d), w.dtype),
    )(w)


@functools.partial(jax.jit, static_argnums=(2, 3))
def _sc_gather(table, idx2, n_rows, d):
    rows_per_w = n_rows // _NW
    n_chunks = rows_per_w // _IDXW
    irows_per_w = n_chunks
    assert n_chunks % _NBUF == 0 and n_chunks >= 2 * _NBUF

    mesh = plsc.VectorSubcoreMesh(core_axis_name="c", subcore_axis_name="s")

    sems = [pltpu.SemaphoreType.DMA] * (2 * _NBUF)

    @functools.partial(
        pl.kernel,
        mesh=mesh,
        out_type=jax.ShapeDtypeStruct((n_rows, d), jnp.float32),
        scratch_types=[
            pltpu.VMEM((irows_per_w, _IDXW), jnp.int32),
            pltpu.VMEM((_NBUF, _IDXW, d), jnp.float32),
        ] + sems,
    )
    def k(table_hbm, idx_hbm, out_hbm, idx_all, rows_v, *all_sems):
        g_sem = all_sems[:_NBUF]
        o_sem = all_sems[_NBUF:]
        wid = lax.axis_index("s") * _NC + lax.axis_index("c")
        irow0 = wid * irows_per_w

        pltpu.sync_copy(idx_hbm.at[pl.ds(irow0, irows_per_w)], idx_all)

        def fire_gather(b, c):
            pltpu.async_copy(
                table_hbm.at[idx_all.at[c]], rows_v.at[b], g_sem[b]
            )

        def wait_gather(b):
            pltpu.make_async_copy(
                table_hbm.at[idx_all.at[0]], rows_v.at[b], g_sem[b]
            ).wait()

        def fire_out(b, c):
            pltpu.async_copy(
                rows_v.at[b],
                out_hbm.at[pl.ds((irow0 + c) * _IDXW, _IDXW)],
                o_sem[b],
            )

        def wait_out(b):
            pltpu.make_async_copy(
                rows_v.at[b], out_hbm.at[pl.ds(0, _IDXW)], o_sem[b]
            ).wait()

        scale = float(d) ** 0.5

        def relu_scale(b):
            def row_fn(r, carry):
                for j in range(d // 16):
                    v = rows_v[b, r, pl.ds(j * 16, 16)]
                    rows_v[b, r, pl.ds(j * 16, 16)] = (
                        jnp.maximum(v, 0.0) * scale
                    )
                return carry

            lax.fori_loop(0, _IDXW, row_fn, 0)

        def step(c, k_static, fire_g, wait_g, wait_o):
            if wait_o:
                wait_out(k_static)
            if fire_g:
                fire_gather(k_static, c)
            if wait_g:
                b2 = (k_static - _DELAY) % _NBUF
                wait_gather(b2)
                relu_scale(b2)
                fire_out(b2, c - _DELAY)

        for c in range(_NBUF):
            step(c, c, True, c >= _DELAY, False)

        def body(i, carry):
            for kk in range(_NBUF):
                step(_NBUF + i * _NBUF + kk, kk, True, True, True)
            return carry

        lax.fori_loop(0, n_chunks // _NBUF - 1, body, 0)

        for c in range(n_chunks, n_chunks + _DELAY):
            step(c, c % _NBUF, False, True, False)
        for b in range(_NBUF):
            wait_out(b)

    return k(table, idx2)


def kernel(x, embed_weight):
    b, h = x.shape
    v, d = embed_weight.shape
    n_rows = b * h
    idx2 = x.reshape(n_rows // _IDXW, _IDXW).astype(jnp.int32)
    out = _sc_gather(embed_weight, idx2, n_rows, d)
    return out.reshape(b, h, d)
